# baseline (speedup 1.0000x reference)
import functools

import jax
import jax.numpy as jnp
from jax import lax
from jax.experimental import pallas as pl
from jax.experimental.pallas import tpu as pltpu
from jax.experimental.pallas import tpu_sc as plsc

B = 16384
D = 64
D2 = 128
NW = 32
BPW = B // NW
CH = 128
NCH = BPW // CH
BT = 2048
CT = 16384


def _rne16(u):
    return u + 0x7FFF + ((u >> 16) & 1)


def _packT_body(ta, tb, out):
    t = jnp.concatenate((ta[...].T, tb[...].T), axis=1)
    y = lax.bitcast_convert_type(t, jnp.int32)
    lo = _rne16(y[:CT // 2])
    hi = _rne16(y[CT // 2:])
    out[...] = ((lo >> 16) & 0xFFFF) | (hi & ~0xFFFF)


@functools.lru_cache(maxsize=4)
def _get_packT(n):
    return pl.pallas_call(
        _packT_body,
        grid=(pl.cdiv(n, CT),),
        in_specs=[
            pl.BlockSpec((D, CT), lambda i: (0, i)),
            pl.BlockSpec((D, CT), lambda i: (0, i)),
        ],
        out_specs=pl.BlockSpec((CT // 2, D2), lambda i: (i, 0)),
        out_shape=jax.ShapeDtypeStruct((pl.cdiv(n, CT) * (CT // 2), D2),
                                       jnp.int32),
    )


def _gather1_body(idx_hbm, table, out, idx_v, buf0, buf1, sem0, sem1):
    wid = lax.axis_index("s") * 2 + lax.axis_index("c")
    pltpu.sync_copy(idx_hbm.at[wid], idx_v)
    base = wid * BPW

    bufs = (buf0, buf1)
    sems = (sem0, sem1)
    prev = pltpu.async_copy(table.at[idx_v.at[0]], bufs[0], sems[0])
    for j in range(1, NCH):
        cur = pltpu.async_copy(table.at[idx_v.at[j]], bufs[j % 2], sems[j % 2])
        prev.wait()
        pltpu.sync_copy(bufs[(j - 1) % 2], out.at[pl.ds(base + (j - 1) * CH, CH)])
        prev = cur
    prev.wait()
    pltpu.sync_copy(bufs[(NCH - 1) % 2], out.at[pl.ds(base + (NCH - 1) * CH, CH)])


@functools.lru_cache(maxsize=1)
def _get_gather1():
    mesh = plsc.VectorSubcoreMesh(core_axis_name="c", subcore_axis_name="s")
    return pl.kernel(
        _gather1_body,
        mesh=mesh,
        out_type=jax.ShapeDtypeStruct((B, D2), jnp.int32),
        scratch_types=[
            pltpu.VMEM((NCH, CH), jnp.int32),
            pltpu.VMEM((CH, D2), jnp.int32),
            pltpu.VMEM((CH, D2), jnp.int32),
            pltpu.SemaphoreType.DMA,
            pltpu.SemaphoreType.DMA,
        ],
    )


def _unpack_half(x, par):
    sel = jnp.where(par > 0, x & ~0xFFFF, x << 16)
    return lax.bitcast_convert_type(sel, jnp.float32)


def _fuse_body(u, i, up, ip, w1a, w1b, b1, wg, wm, bo, out):
    uf = _unpack_half(u[...], up[...])
    if_ = _unpack_half(i[...], ip[...])
    ug = uf[:, :D]
    um = uf[:, D:]
    ig = if_[:, :D]
    im = if_[:, D:]
    h = jnp.dot(um, w1a[...], preferred_element_type=jnp.float32)
    h = h + jnp.dot(im, w1b[...], preferred_element_type=jnp.float32)
    h = jnp.maximum(h + b1[...], 0.0)
    g = ug * ig
    out[...] = (jnp.dot(g, wg[...], preferred_element_type=jnp.float32)
                + jnp.dot(h, wm[...], preferred_element_type=jnp.float32)
                + bo[...])


@functools.lru_cache(maxsize=1)
def _get_fuse():
    return pl.pallas_call(
        _fuse_body,
        grid=(B // BT,),
        in_specs=[
            pl.BlockSpec((BT, D2), lambda i: (i, 0)),
            pl.BlockSpec((BT, D2), lambda i: (i, 0)),
            pl.BlockSpec((BT, 1), lambda i: (i, 0)),
            pl.BlockSpec((BT, 1), lambda i: (i, 0)),
            pl.BlockSpec((D, D), lambda i: (0, 0)),
            pl.BlockSpec((D, D), lambda i: (0, 0)),
            pl.BlockSpec((1, D), lambda i: (0, 0)),
            pl.BlockSpec((D, 1), lambda i: (0, 0)),
            pl.BlockSpec((D, 1), lambda i: (0, 0)),
            pl.BlockSpec((1, 1), lambda i: (0, 0)),
        ],
        out_specs=pl.BlockSpec((BT, 1), lambda i: (i, 0)),
        out_shape=jax.ShapeDtypeStruct((B, 1), jnp.float32),
    )


def kernel(user_indices, item_indices, user_emb_ncf, item_emb_ncf,
           user_emb_mlp, item_emb_mlp, W1, b1, W_out, b_out):
    ui = user_indices.astype(jnp.int32)
    ii = item_indices.astype(jnp.int32)
    H = CT // 2
    ujdx = ((ui // CT) * H + (ui % H)).reshape(NW, NCH, CH)
    ijdx = ((ii // CT) * H + (ii % H)).reshape(NW, NCH, CH)
    upar = ((ui % CT) // H).reshape(B, 1)
    ipar = ((ii % CT) // H).reshape(B, 1)
    nu = user_emb_ncf.shape[0]
    ni = item_emb_ncf.shape[0]
    t_i = _get_packT(ni)(item_emb_ncf.T, item_emb_mlp.T)
    g = _get_gather1()
    i = g(ijdx, t_i)
    t_u = _get_packT(nu)(user_emb_ncf.T, user_emb_mlp.T)
    u = g(ujdx, t_u)
    return _get_fuse()(u, i, upar, ipar, W1[:D], W1[D:], b1.reshape(1, D),
                       W_out[:D], W_out[D:], b_out.reshape(1, 1))


# speedup vs baseline: 1.4717x; 1.4717x over previous
import functools

import jax
import jax.numpy as jnp
from jax import lax
from jax.experimental import pallas as pl
from jax.experimental.pallas import tpu as pltpu
from jax.experimental.pallas import tpu_sc as plsc

B = 16384
D = 64
D2 = 128
NW = 32
BPW = B // NW
CH = 128
NCH = BPW // CH
BT = 2048
CT = 16384


def _rne16(u):
    return u + 0x7FFF + ((u >> 16) & 1)


def _pack_pairs(y):
    h = CT // 2
    return ((_rne16(y[:, :h]) >> 16) & 0xFFFF) | (_rne16(y[:, h:]) & ~0xFFFF)


def _packT_body(ta, tb, out):
    wa = _pack_pairs(lax.bitcast_convert_type(ta[...], jnp.int32))
    wb = _pack_pairs(lax.bitcast_convert_type(tb[...], jnp.int32))
    out[...] = jnp.concatenate((wa.T, wb.T), axis=1)


@functools.lru_cache(maxsize=4)
def _get_packT(n):
    return pl.pallas_call(
        _packT_body,
        grid=(pl.cdiv(n, CT),),
        in_specs=[
            pl.BlockSpec((D, CT), lambda i: (0, i)),
            pl.BlockSpec((D, CT), lambda i: (0, i)),
        ],
        out_specs=pl.BlockSpec((CT // 2, D2), lambda i: (i, 0)),
        out_shape=jax.ShapeDtypeStruct((pl.cdiv(n, CT) * (CT // 2), D2),
                                       jnp.int32),
    )


def _gather1_body(idx_hbm, table, out, idx_v, buf0, buf1, sem0, sem1):
    wid = lax.axis_index("s") * 2 + lax.axis_index("c")
    pltpu.sync_copy(idx_hbm.at[wid], idx_v)
    base = wid * BPW

    bufs = (buf0, buf1)
    sems = (sem0, sem1)
    prev = pltpu.async_copy(table.at[idx_v.at[0]], bufs[0], sems[0])
    for j in range(1, NCH):
        cur = pltpu.async_copy(table.at[idx_v.at[j]], bufs[j % 2], sems[j % 2])
        prev.wait()
        pltpu.sync_copy(bufs[(j - 1) % 2], out.at[pl.ds(base + (j - 1) * CH, CH)])
        prev = cur
    prev.wait()
    pltpu.sync_copy(bufs[(NCH - 1) % 2], out.at[pl.ds(base + (NCH - 1) * CH, CH)])


@functools.lru_cache(maxsize=1)
def _get_gather1():
    mesh = plsc.VectorSubcoreMesh(core_axis_name="c", subcore_axis_name="s")
    return pl.kernel(
        _gather1_body,
        mesh=mesh,
        out_type=jax.ShapeDtypeStruct((B, D2), jnp.int32),
        scratch_types=[
            pltpu.VMEM((NCH, CH), jnp.int32),
            pltpu.VMEM((CH, D2), jnp.int32),
            pltpu.VMEM((CH, D2), jnp.int32),
            pltpu.SemaphoreType.DMA,
            pltpu.SemaphoreType.DMA,
        ],
    )


def _unpack_half(x, par):
    sel = jnp.where(par > 0, x & ~0xFFFF, x << 16)
    return lax.bitcast_convert_type(sel, jnp.float32)


def _fuse_body(u, i, up, ip, w1a, w1b, b1, wg, wm, bo, out):
    uf = _unpack_half(u[...], up[...])
    if_ = _unpack_half(i[...], ip[...])
    ug = uf[:, :D]
    um = uf[:, D:]
    ig = if_[:, :D]
    im = if_[:, D:]
    h = jnp.dot(um, w1a[...], preferred_element_type=jnp.float32)
    h = h + jnp.dot(im, w1b[...], preferred_element_type=jnp.float32)
    h = jnp.maximum(h + b1[...], 0.0)
    g = ug * ig
    out[...] = (jnp.dot(g, wg[...], preferred_element_type=jnp.float32)
                + jnp.dot(h, wm[...], preferred_element_type=jnp.float32)
                + bo[...])


@functools.lru_cache(maxsize=1)
def _get_fuse():
    return pl.pallas_call(
        _fuse_body,
        grid=(B // BT,),
        in_specs=[
            pl.BlockSpec((BT, D2), lambda i: (i, 0)),
            pl.BlockSpec((BT, D2), lambda i: (i, 0)),
            pl.BlockSpec((BT, 1), lambda i: (i, 0)),
            pl.BlockSpec((BT, 1), lambda i: (i, 0)),
            pl.BlockSpec((D, D), lambda i: (0, 0)),
            pl.BlockSpec((D, D), lambda i: (0, 0)),
            pl.BlockSpec((1, D), lambda i: (0, 0)),
            pl.BlockSpec((D, 1), lambda i: (0, 0)),
            pl.BlockSpec((D, 1), lambda i: (0, 0)),
            pl.BlockSpec((1, 1), lambda i: (0, 0)),
        ],
        out_specs=pl.BlockSpec((BT, 1), lambda i: (i, 0)),
        out_shape=jax.ShapeDtypeStruct((B, 1), jnp.float32),
    )


def kernel(user_indices, item_indices, user_emb_ncf, item_emb_ncf,
           user_emb_mlp, item_emb_mlp, W1, b1, W_out, b_out):
    ui = user_indices.astype(jnp.int32)
    ii = item_indices.astype(jnp.int32)
    H = CT // 2
    ujdx = ((ui // CT) * H + (ui % H)).reshape(NW, NCH, CH)
    ijdx = ((ii // CT) * H + (ii % H)).reshape(NW, NCH, CH)
    upar = ((ui % CT) // H).reshape(B, 1)
    ipar = ((ii % CT) // H).reshape(B, 1)
    nu = user_emb_ncf.shape[0]
    ni = item_emb_ncf.shape[0]
    t_i = _get_packT(ni)(item_emb_ncf.T, item_emb_mlp.T)
    g = _get_gather1()
    i = g(ijdx, t_i)
    t_u = _get_packT(nu)(user_emb_ncf.T, user_emb_mlp.T)
    u = g(ujdx, t_u)
    return _get_fuse()(u, i, upar, ipar, W1[:D], W1[D:], b1.reshape(1, D),
                       W_out[:D], W_out[D:], b_out.reshape(1, 1))
